# Initial kernel scaffold; baseline (speedup 1.0000x reference)
import jax
import jax.numpy as jnp
from jax.experimental import pallas as pl


def kernel(features, edge_index, W, a_src, a_dst):
    raise NotImplementedError("write your pallas kernel here")


# speedup vs baseline: 30.6488x; 30.6488x over previous
import functools

import jax
import jax.numpy as jnp
from jax import lax
from jax.experimental import pallas as pl
from jax.experimental.pallas import tpu as pltpu
from jax.experimental.pallas import tpu_sc as plsc

NCORES = 2
NSUB = 16
LANES = 16
CHUNK = 80
BM = 1000
NUM_LAYERS = 3


def _elu(v):
    return jnp.where(v > 0, v, jnp.exp(v) - 1.0)


def _alphas_and_gmax(i, h, a_src_ref, a_dst_ref, as_ref, ad_ref, g_ref):
    as_blk = jnp.dot(h, a_src_ref[0, :], preferred_element_type=jnp.float32)
    ad_blk = jnp.dot(h, a_dst_ref[0, :], preferred_element_type=jnp.float32)
    as_ref[0, 0, :] = as_blk
    ad_ref[0, 0, :] = ad_blk

    @pl.when(i == 0)
    def _():
        g_ref[...] = jnp.full((8, 128), -jnp.inf, jnp.float32)

    g_ref[...] = jnp.maximum(g_ref[...], jnp.full((8, 128), jnp.max(as_blk)))


def _prep_x_body(x_ref, w_ref, a_src_ref, a_dst_ref, h_ref, as_ref, ad_ref, g_ref):
    i = pl.program_id(0)
    h = jnp.dot(x_ref[...], w_ref[...], preferred_element_type=jnp.float32)
    h_ref[...] = h
    _alphas_and_gmax(i, h, a_src_ref, a_dst_ref, as_ref, ad_ref, g_ref)


def _prep_p_body(p_ref0, p_ref1, d_ref0, d_ref1, w_ref, a_src_ref, a_dst_ref,
                 h_ref, as_ref, ad_ref, g_ref):
    i = pl.program_id(0)
    num = p_ref0[0, :, :] + p_ref1[0, :, :]
    den = d_ref0[0, :, :] + d_ref1[0, :, :] + 1e-16
    x = _elu(num / den)
    h = jnp.dot(x, w_ref[...], preferred_element_type=jnp.float32)
    h_ref[...] = h
    _alphas_and_gmax(i, h, a_src_ref, a_dst_ref, as_ref, ad_ref, g_ref)


def _combine_body(p_ref0, p_ref1, d_ref0, d_ref1, o_ref):
    num = p_ref0[0, :, :] + p_ref1[0, :, :]
    den = d_ref0[0, :, :] + d_ref1[0, :, :] + 1e-16
    o_ref[...] = _elu(num / den)


def _make_tc_kernels(n, d, np_pad):
    nb = n // BM
    w_spec = pl.BlockSpec((d, d), lambda i: (0, 0))
    a_spec = pl.BlockSpec((1, d), lambda i: (0, 0))
    x_spec = pl.BlockSpec((BM, d), lambda i: (i, 0))
    p0_spec = pl.BlockSpec((1, BM, d), lambda i: (0, i, 0))
    p1_spec = pl.BlockSpec((1, BM, d), lambda i: (1, i, 0))
    d0_spec = pl.BlockSpec((1, BM, 1), lambda i: (0, i, 0))
    d1_spec = pl.BlockSpec((1, BM, 1), lambda i: (1, i, 0))
    al_spec = pl.BlockSpec((1, 1, BM), lambda i: (i, 0, 0))
    g_spec = pl.BlockSpec((8, 128), lambda i: (0, 0))

    out_types = [
        jax.ShapeDtypeStruct((n, d), jnp.float32),
        jax.ShapeDtypeStruct((nb, 1, BM), jnp.float32),
        jax.ShapeDtypeStruct((nb, 1, BM), jnp.float32),
        jax.ShapeDtypeStruct((8, 128), jnp.float32),
    ]
    out_specs = [x_spec, al_spec, al_spec, g_spec]

    prep_x = pl.pallas_call(
        _prep_x_body,
        grid=(nb,),
        in_specs=[x_spec, w_spec, a_spec, a_spec],
        out_specs=out_specs,
        out_shape=out_types,
    )
    prep_p = pl.pallas_call(
        _prep_p_body,
        grid=(nb,),
        in_specs=[p0_spec, p1_spec, d0_spec, d1_spec, w_spec, a_spec, a_spec],
        out_specs=out_specs,
        out_shape=out_types,
    )
    combine = pl.pallas_call(
        _combine_body,
        grid=(nb,),
        in_specs=[p0_spec, p1_spec, d0_spec, d1_spec],
        out_specs=x_spec,
        out_shape=jax.ShapeDtypeStruct((n, d), jnp.float32),
    )
    return prep_x, prep_p, combine


GRP = 25


def _make_sc_edge(n, d, e, np_pad):
    per_tile = e // (NCORES * NSUB)
    ngrp = per_tile // (GRP * CHUNK)
    assert ngrp * GRP * CHUNK * NCORES * NSUB == e
    rows_per_tile = np_pad // NSUB
    nzc = rows_per_tile // CHUNK
    assert nzc * CHUNK == rows_per_tile

    mesh = plsc.VectorSubcoreMesh(
        core_axis_name="c", subcore_axis_name="s",
        num_cores=NCORES, num_subcores=NSUB)

    @functools.partial(
        pl.kernel,
        out_type=[
            jax.ShapeDtypeStruct((NCORES, np_pad, d), jnp.float32),
            jax.ShapeDtypeStruct((NCORES, np_pad), jnp.float32),
        ],
        mesh=mesh,
        compiler_params=pltpu.CompilerParams(needs_layout_passes=False),
        scratch_types=[
            pltpu.VMEM((n,), jnp.float32),
            pltpu.VMEM((n,), jnp.float32),
            pltpu.VMEM((1, 128), jnp.float32),
            pltpu.VMEM((GRP, CHUNK), jnp.int32),
            pltpu.VMEM((GRP, CHUNK), jnp.int32),
            pltpu.VMEM((CHUNK, d), jnp.float32),
            pltpu.VMEM((CHUNK,), jnp.float32),
            pltpu.VMEM_SHARED((np_pad, d), jnp.float32),
            pltpu.VMEM_SHARED((np_pad,), jnp.float32),
        ],
    )
    def sc_edge(h_hbm, as_hbm, ad_hbm, g_hbm, src_hbm, dst_hbm,
                p_out, den_out, as_v, ad_v, g_v, src_v, dst_v, rows_v, ee_v,
                p_sp, d_sp):
        cid = lax.axis_index("c")
        sid = lax.axis_index("s")
        row0 = sid * rows_per_tile

        def _zrows(i, _):
            for u in range(d // LANES):
                rows_v[i, pl.ds(u * LANES, LANES)] = jnp.zeros((LANES,), jnp.float32)
            return 0
        lax.fori_loop(0, CHUNK, _zrows, 0)
        for u in range(CHUNK // LANES):
            ee_v[pl.ds(u * LANES, LANES)] = jnp.zeros((LANES,), jnp.float32)
        for b in range(nzc):
            pltpu.sync_copy(rows_v, p_sp.at[pl.ds(row0 + b * CHUNK, CHUNK)])
            pltpu.sync_copy(ee_v, d_sp.at[pl.ds(row0 + b * CHUNK, CHUNK)])

        pltpu.sync_copy(as_hbm, as_v)
        pltpu.sync_copy(ad_hbm, ad_v)
        pltpu.sync_copy(g_hbm, g_v)
        plsc.subcore_barrier()

        g16 = g_v[0, pl.ds(0, LANES)]

        def group(gi, _):
            pltpu.sync_copy(src_hbm.at[cid, sid, gi], src_v)
            pltpu.sync_copy(dst_hbm.at[cid, sid, gi], dst_v)

            def body(j, _):
                pltpu.sync_copy(h_hbm.at[src_v.at[j]], rows_v)
                for q in range(CHUNK // LANES):
                    sl = pl.ds(q * LANES, LANES)
                    s16 = src_v[j, sl]
                    d16 = dst_v[j, sl]
                    a_s = plsc.load_gather(as_v, [s16])
                    a_d = plsc.load_gather(ad_v, [d16])
                    s = a_s + a_d
                    ee = jnp.exp(jnp.where(s > 0, s, 0.2 * s)
                                 - jnp.where(a_d + g16 > 0, a_d + g16,
                                             0.2 * (a_d + g16)))
                    ee_v[sl] = ee

                def scale(q, _):
                    ee16 = ee_v[pl.ds(q * LANES, LANES)]
                    base = q * LANES
                    for r in range(LANES):
                        cf = jnp.full((LANES,), ee16[r], jnp.float32)
                        for u in range(d // LANES):
                            sl = pl.ds(u * LANES, LANES)
                            rows_v[base + r, sl] = rows_v[base + r, sl] * cf
                    return 0
                lax.fori_loop(0, CHUNK // LANES, scale, 0)

                pltpu.sync_copy(rows_v, p_sp.at[dst_v.at[j]], add=True)
                pltpu.sync_copy(ee_v, d_sp.at[dst_v.at[j]], add=True)
                return 0
            lax.fori_loop(0, GRP, body, 0)
            return 0
        lax.fori_loop(0, ngrp, group, 0)
        plsc.subcore_barrier()

        pltpu.sync_copy(p_sp.at[pl.ds(row0, rows_per_tile)],
                        p_out.at[cid, pl.ds(row0, rows_per_tile)])
        pltpu.sync_copy(d_sp.at[pl.ds(row0, rows_per_tile)],
                        den_out.at[cid, pl.ds(row0, rows_per_tile)])

    return sc_edge


def kernel(features, edge_index, W, a_src, a_dst):
    n, d = features.shape
    e = edge_index.shape[1]
    np_pad = ((n + NSUB * CHUNK - 1) // (NSUB * CHUNK)) * (NSUB * CHUNK)

    prep_x, prep_p, combine = _make_tc_kernels(n, d, np_pad)
    sc_edge = _make_sc_edge(n, d, e, np_pad)

    ngrp = e // (NCORES * NSUB * GRP * CHUNK)
    src4 = edge_index[0].reshape(NCORES, NSUB, ngrp, GRP, CHUNK)
    dst4 = edge_index[1].reshape(NCORES, NSUB, ngrp, GRP, CHUNK)
    a_src2 = a_src.reshape(1, d)
    a_dst2 = a_dst.reshape(1, d)

    p = dnm = None
    for layer in range(NUM_LAYERS):
        if layer == 0:
            h, as3, ad3, g = prep_x(features, W, a_src2, a_dst2)
        else:
            h, as3, ad3, g = prep_p(p, p, dnm, dnm, W, a_src2, a_dst2)
        pflat, dflat = sc_edge(h, as3.reshape(n), ad3.reshape(n), g[0:1], src4, dst4)
        p = pflat
        dnm = dflat.reshape(NCORES, np_pad, 1)
    return combine(p, p, dnm, dnm)
